# baseline (speedup 1.0000x reference)
import functools

import jax
import jax.numpy as jnp
from jax import lax
from jax.experimental import pallas as pl
from jax.experimental.pallas import tpu as pltpu
from jax.experimental.pallas import tpu_sc as plsc

N = 10000
E = 320000
D = 128

NC = 2
NS = 16
NW = NC * NS
EPT = E // NW
CH = 80
NITER = EPT // CH

STRIPE = 640
LAST = N - 15 * STRIPE
BM = 1000
GRID = N // BM

_MESH = plsc.VectorSubcoreMesh(
    core_axis_name="c", subcore_axis_name="s", num_cores=NC, num_subcores=NS
)


def _stripe_copy(do_copy):
    s = lax.axis_index("s")

    @pl.when(s < 15)
    def _():
        off = pl.multiple_of(s * STRIPE, 8)
        do_copy(off, STRIPE)

    @pl.when(s == 15)
    def _():
        do_copy(15 * STRIPE, LAST)


@functools.partial(
    pl.kernel,
    out_type=jax.ShapeDtypeStruct((2 * N,), jnp.float32),
    mesh=_MESH,
    scratch_types=[
        pltpu.VMEM((EPT,), jnp.int32),
        pltpu.VMEM((N,), jnp.float32),
        pltpu.VMEM((NS * STRIPE,), jnp.float32),
        pltpu.VMEM((STRIPE,), jnp.float32),
        pltpu.VMEM_SHARED((NS * N,), jnp.float32),
        pltpu.SemaphoreType.DMA,
    ],
    compiler_params=pltpu.CompilerParams(needs_layout_passes=False),
)
def _sc_hist(dst_hbm, hist_hbm, didx, hloc, tbuf, obuf, shared, isem):
    c = lax.axis_index("c")
    s = lax.axis_index("s")
    t = c * NS + s
    zero16 = jnp.zeros((16,), jnp.float32)
    ones16 = jnp.ones((16,), jnp.float32)

    base = pl.multiple_of(t * EPT, 8)
    idma = pltpu.async_copy(dst_hbm.at[pl.ds(base, EPT)], didx, isem)

    def z_body(i, carry):
        hloc[pl.ds(i * 16, 16)] = zero16
        return carry

    lax.fori_loop(0, N // 16, z_body, 0)
    idma.wait()

    def e_body(i, carry):
        idx = didx[pl.ds(i * 16, 16)]
        plsc.addupdate_scatter(hloc, [idx], ones16)
        return carry

    lax.fori_loop(0, EPT // 16, e_body, 0)
    pltpu.sync_copy(hloc, shared.at[pl.ds(pl.multiple_of(s * N, 8), N)])
    plsc.subcore_barrier()

    def reduce_stripe(off, size):
        for r in range(NS):
            pltpu.sync_copy(
                shared.at[pl.ds(r * N + off, size)],
                tbuf.at[pl.ds(r * STRIPE, size)],
            )

        def r_body(j, carry):
            acc = tbuf[pl.ds(j * 16, 16)]
            for r in range(1, NS):
                acc = acc + tbuf[pl.ds(r * STRIPE + j * 16, 16)]
            obuf[pl.ds(j * 16, 16)] = acc
            return carry

        lax.fori_loop(0, size // 16, r_body, 0)
        pltpu.sync_copy(
            obuf.at[pl.ds(0, size)], hist_hbm.at[pl.ds(c * N + off, size)]
        )

    _stripe_copy(reduce_stripe)


@functools.partial(
    pl.kernel,
    out_type=jax.ShapeDtypeStruct((2 * N, D), jnp.float32),
    mesh=_MESH,
    scratch_types=[
        pltpu.VMEM((EPT,), jnp.int32),
        pltpu.VMEM((EPT,), jnp.int32),
        pltpu.VMEM((CH, D), jnp.float32),
        pltpu.VMEM((CH, D), jnp.float32),
        pltpu.VMEM_SHARED((N, D), jnp.float32),
        pltpu.SemaphoreType.DMA,
        pltpu.SemaphoreType.DMA,
        pltpu.SemaphoreType.DMA,
        pltpu.SemaphoreType.DMA,
        pltpu.SemaphoreType.DMA,
    ],
)
def _sc_scatter(src_hbm, dst_hbm, z_hbm, zeros_hbm, out_hbm, sidx, didx,
                rows0, rows1, shared, gsem0, gsem1, ssem0, ssem1, isem):
    c = lax.axis_index("c")
    s = lax.axis_index("s")
    t = c * NS + s
    base = pl.multiple_of(t * EPT, 8)
    pltpu.async_copy(src_hbm.at[pl.ds(base, EPT)], sidx, isem).wait()
    idma = pltpu.async_copy(dst_hbm.at[pl.ds(base, EPT)], didx, isem)
    _stripe_copy(
        lambda off, size: pltpu.sync_copy(
            zeros_hbm.at[pl.ds(0, size)], shared.at[pl.ds(off, size)]
        )
    )
    idma.wait()
    plsc.subcore_barrier()

    def gather(i, rows, gsem):
        off = pl.multiple_of(i * CH, 8)
        pltpu.async_copy(z_hbm.at[sidx.at[pl.ds(off, CH)]], rows, gsem)

    def gwait(rows, gsem):
        pltpu.make_async_copy(z_hbm.at[pl.ds(0, CH)], rows, gsem).wait()

    def scatter(i, rows, ssem):
        off = pl.multiple_of(i * CH, 8)
        pltpu.async_copy(rows, shared.at[didx.at[pl.ds(off, CH)]], ssem,
                         add=True)

    def swait(rows, ssem):
        pltpu.make_async_copy(rows, shared.at[pl.ds(0, CH)], ssem).wait()

    gather(0, rows0, gsem0)
    gather(1, rows1, gsem1)

    def body(k, carry):
        i0 = 2 * k
        gwait(rows0, gsem0)
        scatter(i0, rows0, ssem0)
        gwait(rows1, gsem1)
        scatter(i0 + 1, rows1, ssem1)
        swait(rows0, ssem0)
        gather(jnp.minimum(i0 + 2, NITER - 1), rows0, gsem0)
        swait(rows1, ssem1)
        gather(jnp.minimum(i0 + 3, NITER - 1), rows1, gsem1)
        return carry

    lax.fori_loop(0, (NITER - 1) // 2, body, 0)
    gwait(rows0, gsem0)
    scatter(NITER - 1, rows0, ssem0)
    gwait(rows1, gsem1)
    swait(rows0, ssem0)
    plsc.subcore_barrier()
    _stripe_copy(
        lambda off, size: pltpu.sync_copy(
            shared.at[pl.ds(off, size)], out_hbm.at[pl.ds(c * N + off, size)]
        )
    )


def _dinv(h0, h1):
    deg = 1.0 + h0[...] + h1[...]
    return lax.rsqrt(deg)


def _tc_a_body(x_ref, w1_ref, h0_ref, h1_ref, z1_ref):
    dinv = _dinv(h0_ref, h1_ref)
    y = jnp.dot(x_ref[...], w1_ref[...], preferred_element_type=jnp.float32)
    z1_ref[...] = y * dinv


def _tc_b_body(p0_ref, p1_ref, z1_ref, h0_ref, h1_ref, w2_ref, b1_ref, z2_ref):
    dinv = _dinv(h0_ref, h1_ref)
    h = dinv * (p0_ref[...] + p1_ref[...] + z1_ref[...]) + b1_ref[...]
    h = jnp.maximum(h, 0.0)
    y = jnp.dot(h, w2_ref[...], preferred_element_type=jnp.float32)
    z2_ref[...] = y * dinv


def _tc_c_body(q0_ref, q1_ref, z2_ref, h0_ref, h1_ref, b2_ref, out_ref):
    dinv = _dinv(h0_ref, h1_ref)
    o = dinv * (q0_ref[...] + q1_ref[...] + z2_ref[...]) + b2_ref[...]
    m = jnp.max(o, axis=1, keepdims=True)
    ex = jnp.exp(o - m)
    lse = jnp.log(jnp.sum(ex, axis=1, keepdims=True)) + m
    out_ref[...] = o - lse


def _row_spec(shift=0, width=D):
    return pl.BlockSpec((BM, width), lambda i, s=shift: (i + s, 0))


def _full_spec(shape):
    return pl.BlockSpec(shape, lambda i: (0, 0))


def kernel(x, edge_index, W1, b1, W2, b2):
    src = edge_index[0]
    dst = edge_index[1]
    zeros128 = jnp.zeros((STRIPE, D), jnp.float32)
    b1r = b1.reshape(1, D)
    b2r = b2.reshape(1, D)

    hist = _sc_hist(dst).reshape(2 * N, 1)

    z1 = pl.pallas_call(
        _tc_a_body,
        grid=(GRID,),
        in_specs=[
            _row_spec(),
            _full_spec((D, D)),
            _row_spec(width=1),
            _row_spec(shift=GRID, width=1),
        ],
        out_specs=_row_spec(),
        out_shape=jax.ShapeDtypeStruct((N, D), jnp.float32),
    )(x, W1, hist, hist)

    p = _sc_scatter(src, dst, z1, zeros128)

    z2 = pl.pallas_call(
        _tc_b_body,
        grid=(GRID,),
        in_specs=[
            _row_spec(),
            _row_spec(shift=GRID),
            _row_spec(),
            _row_spec(width=1),
            _row_spec(shift=GRID, width=1),
            _full_spec((D, D)),
            _full_spec((1, D)),
        ],
        out_specs=_row_spec(),
        out_shape=jax.ShapeDtypeStruct((N, D), jnp.float32),
    )(p, p, z1, hist, hist, W2, b1r)

    q = _sc_scatter(src, dst, z2, zeros128)

    out = pl.pallas_call(
        _tc_c_body,
        grid=(GRID,),
        in_specs=[
            _row_spec(),
            _row_spec(shift=GRID),
            _row_spec(),
            _row_spec(width=1),
            _row_spec(shift=GRID, width=1),
            _full_spec((1, D)),
        ],
        out_specs=_row_spec(),
        out_shape=jax.ShapeDtypeStruct((N, D), jnp.float32),
    )(q, q, z2, hist, hist, b2r)

    return out


# speedup vs baseline: 1.2384x; 1.2384x over previous
import functools

import jax
import jax.numpy as jnp
from jax import lax
from jax.experimental import pallas as pl
from jax.experimental.pallas import tpu as pltpu
from jax.experimental.pallas import tpu_sc as plsc

N = 10000
E = 320000
D = 128

NC = 2
NS = 16
NW = NC * NS
EPT = E // NW
CH = 80
NITER = EPT // CH

STRIPE = 640
LAST = N - 15 * STRIPE
BM = 2000
GRID = N // BM

_MESH = plsc.VectorSubcoreMesh(
    core_axis_name="c", subcore_axis_name="s", num_cores=NC, num_subcores=NS
)


def _stripe_copy(do_copy):
    s = lax.axis_index("s")

    @pl.when(s < 15)
    def _():
        off = pl.multiple_of(s * STRIPE, 8)
        do_copy(off, STRIPE)

    @pl.when(s == 15)
    def _():
        do_copy(15 * STRIPE, LAST)


@functools.partial(
    pl.kernel,
    out_type=jax.ShapeDtypeStruct((2 * N,), jnp.float32),
    mesh=_MESH,
    scratch_types=[
        pltpu.VMEM((EPT,), jnp.int32),
        pltpu.VMEM((N,), jnp.float32),
        pltpu.VMEM((NS * STRIPE,), jnp.float32),
        pltpu.VMEM((STRIPE,), jnp.float32),
        pltpu.VMEM_SHARED((NS * N,), jnp.float32),
        pltpu.SemaphoreType.DMA,
    ],
    compiler_params=pltpu.CompilerParams(needs_layout_passes=False),
)
def _sc_hist(dst_hbm, hist_hbm, didx, hloc, tbuf, obuf, shared, isem):
    c = lax.axis_index("c")
    s = lax.axis_index("s")
    t = c * NS + s
    zero16 = jnp.zeros((16,), jnp.float32)
    ones16 = jnp.ones((16,), jnp.float32)

    base = pl.multiple_of(t * EPT, 8)
    idma = pltpu.async_copy(dst_hbm.at[pl.ds(base, EPT)], didx, isem)

    def z_body(i, carry):
        hloc[pl.ds(i * 16, 16)] = zero16
        return carry

    lax.fori_loop(0, N // 16, z_body, 0)
    idma.wait()

    def e_body(i, carry):
        idx = didx[pl.ds(i * 16, 16)]
        plsc.addupdate_scatter(hloc, [idx], ones16)
        return carry

    lax.fori_loop(0, EPT // 16, e_body, 0)
    pltpu.sync_copy(hloc, shared.at[pl.ds(pl.multiple_of(s * N, 8), N)])
    plsc.subcore_barrier()

    def reduce_stripe(off, size):
        for r in range(NS):
            pltpu.sync_copy(
                shared.at[pl.ds(r * N + off, size)],
                tbuf.at[pl.ds(r * STRIPE, size)],
            )

        def r_body(j, carry):
            acc = tbuf[pl.ds(j * 16, 16)]
            for r in range(1, NS):
                acc = acc + tbuf[pl.ds(r * STRIPE + j * 16, 16)]
            obuf[pl.ds(j * 16, 16)] = acc
            return carry

        lax.fori_loop(0, size // 16, r_body, 0)
        pltpu.sync_copy(
            obuf.at[pl.ds(0, size)], hist_hbm.at[pl.ds(c * N + off, size)]
        )

    _stripe_copy(reduce_stripe)


@functools.partial(
    pl.kernel,
    out_type=jax.ShapeDtypeStruct((2 * N, D), jnp.float32),
    mesh=_MESH,
    scratch_types=[
        pltpu.VMEM((EPT,), jnp.int32),
        pltpu.VMEM((EPT,), jnp.int32),
        pltpu.VMEM((CH, D), jnp.float32),
        pltpu.VMEM((CH, D), jnp.float32),
        pltpu.VMEM_SHARED((N, D), jnp.float32),
        pltpu.SemaphoreType.DMA,
        pltpu.SemaphoreType.DMA,
        pltpu.SemaphoreType.DMA,
    ],
)
def _sc_scatter(src_hbm, dst_hbm, z_hbm, zeros_hbm, out_hbm, sidx, didx,
                rows0, rows1, shared, gsem0, gsem1, isem):
    c = lax.axis_index("c")
    s = lax.axis_index("s")
    t = c * NS + s
    base = pl.multiple_of(t * EPT, 8)
    pltpu.async_copy(src_hbm.at[pl.ds(base, EPT)], sidx, isem).wait()
    idma = pltpu.async_copy(dst_hbm.at[pl.ds(base, EPT)], didx, isem)
    _stripe_copy(
        lambda off, size: pltpu.sync_copy(
            zeros_hbm.at[pl.ds(0, size)], shared.at[pl.ds(off, size)]
        )
    )
    idma.wait()
    plsc.subcore_barrier()

    def gather(i, rows, gsem):
        off = pl.multiple_of(i * CH, 8)
        pltpu.async_copy(z_hbm.at[sidx.at[pl.ds(off, CH)]], rows, gsem)

    def gwait(rows, gsem):
        pltpu.make_async_copy(z_hbm.at[pl.ds(0, CH)], rows, gsem).wait()

    def scatter(i, rows):
        off = pl.multiple_of(i * CH, 8)
        pltpu.sync_copy(rows, shared.at[didx.at[pl.ds(off, CH)]], add=True)

    gather(0, rows0, gsem0)
    gather(1, rows1, gsem1)

    def body(k, carry):
        i0 = 2 * k
        gwait(rows0, gsem0)
        scatter(i0, rows0)
        gather(jnp.minimum(i0 + 2, NITER - 1), rows0, gsem0)
        gwait(rows1, gsem1)
        scatter(i0 + 1, rows1)
        gather(jnp.minimum(i0 + 3, NITER - 1), rows1, gsem1)
        return carry

    lax.fori_loop(0, (NITER - 1) // 2, body, 0)
    gwait(rows0, gsem0)
    scatter(NITER - 1, rows0)
    gwait(rows1, gsem1)
    plsc.subcore_barrier()
    _stripe_copy(
        lambda off, size: pltpu.sync_copy(
            shared.at[pl.ds(off, size)], out_hbm.at[pl.ds(c * N + off, size)]
        )
    )


def _dinv(h0, h1):
    deg = 1.0 + h0[...] + h1[...]
    return lax.rsqrt(deg)


def _tc_a_body(x_ref, w1_ref, h0_ref, h1_ref, z1_ref):
    dinv = _dinv(h0_ref, h1_ref)
    y = jnp.dot(x_ref[...], w1_ref[...], preferred_element_type=jnp.float32)
    z1_ref[...] = y * dinv


def _tc_b_body(p0_ref, p1_ref, z1_ref, h0_ref, h1_ref, w2_ref, b1_ref, z2_ref):
    dinv = _dinv(h0_ref, h1_ref)
    h = dinv * (p0_ref[...] + p1_ref[...] + z1_ref[...]) + b1_ref[...]
    h = jnp.maximum(h, 0.0)
    y = jnp.dot(h, w2_ref[...], preferred_element_type=jnp.float32)
    z2_ref[...] = y * dinv


def _tc_c_body(q0_ref, q1_ref, z2_ref, h0_ref, h1_ref, b2_ref, out_ref):
    dinv = _dinv(h0_ref, h1_ref)
    o = dinv * (q0_ref[...] + q1_ref[...] + z2_ref[...]) + b2_ref[...]
    m = jnp.max(o, axis=1, keepdims=True)
    ex = jnp.exp(o - m)
    lse = jnp.log(jnp.sum(ex, axis=1, keepdims=True)) + m
    out_ref[...] = o - lse


def _row_spec(shift=0, width=D):
    return pl.BlockSpec((BM, width), lambda i, s=shift: (i + s, 0))


def _full_spec(shape):
    return pl.BlockSpec(shape, lambda i: (0, 0))


def kernel(x, edge_index, W1, b1, W2, b2):
    src = edge_index[0]
    dst = edge_index[1]
    zeros128 = jnp.zeros((STRIPE, D), jnp.float32)
    b1r = b1.reshape(1, D)
    b2r = b2.reshape(1, D)

    hist = _sc_hist(dst).reshape(2 * N, 1)

    z1 = pl.pallas_call(
        _tc_a_body,
        grid=(GRID,),
        in_specs=[
            _row_spec(),
            _full_spec((D, D)),
            _row_spec(width=1),
            _row_spec(shift=GRID, width=1),
        ],
        out_specs=_row_spec(),
        out_shape=jax.ShapeDtypeStruct((N, D), jnp.float32),
    )(x, W1, hist, hist)

    p = _sc_scatter(src, dst, z1, zeros128)

    z2 = pl.pallas_call(
        _tc_b_body,
        grid=(GRID,),
        in_specs=[
            _row_spec(),
            _row_spec(shift=GRID),
            _row_spec(),
            _row_spec(width=1),
            _row_spec(shift=GRID, width=1),
            _full_spec((D, D)),
            _full_spec((1, D)),
        ],
        out_specs=_row_spec(),
        out_shape=jax.ShapeDtypeStruct((N, D), jnp.float32),
    )(p, p, z1, hist, hist, W2, b1r)

    q = _sc_scatter(src, dst, z2, zeros128)

    out = pl.pallas_call(
        _tc_c_body,
        grid=(GRID,),
        in_specs=[
            _row_spec(),
            _row_spec(shift=GRID),
            _row_spec(),
            _row_spec(width=1),
            _row_spec(shift=GRID, width=1),
            _full_spec((1, D)),
        ],
        out_specs=_row_spec(),
        out_shape=jax.ShapeDtypeStruct((N, D), jnp.float32),
    )(q, q, z2, hist, hist, b2r)

    return out


# speedup vs baseline: 1.2743x; 1.0290x over previous
import functools

import jax
import jax.numpy as jnp
from jax import lax
from jax.experimental import pallas as pl
from jax.experimental.pallas import tpu as pltpu
from jax.experimental.pallas import tpu_sc as plsc

N = 10000
E = 320000
D = 128

NC = 2
NS = 16
NW = NC * NS
EPT = E // NW
CH = 80
NITER = EPT // CH

STRIPE = 640
LAST = N - 15 * STRIPE
BM = 2000
GRID = N // BM

_MESH = plsc.VectorSubcoreMesh(
    core_axis_name="c", subcore_axis_name="s", num_cores=NC, num_subcores=NS
)


def _stripe_copy(do_copy):
    s = lax.axis_index("s")

    @pl.when(s < 15)
    def _():
        off = pl.multiple_of(s * STRIPE, 8)
        do_copy(off, STRIPE)

    @pl.when(s == 15)
    def _():
        do_copy(15 * STRIPE, LAST)


@functools.partial(
    pl.kernel,
    out_type=jax.ShapeDtypeStruct((2 * N,), jnp.float32),
    mesh=_MESH,
    scratch_types=[
        pltpu.VMEM((EPT,), jnp.int32),
        pltpu.VMEM((N,), jnp.float32),
        pltpu.VMEM((NS * STRIPE,), jnp.float32),
        pltpu.VMEM((STRIPE,), jnp.float32),
        pltpu.VMEM_SHARED((NS * N,), jnp.float32),
        pltpu.SemaphoreType.DMA,
    ],
    compiler_params=pltpu.CompilerParams(needs_layout_passes=False),
)
def _sc_hist(edge_hbm, hist_hbm, didx, hloc, tbuf, obuf, shared, isem):
    c = lax.axis_index("c")
    s = lax.axis_index("s")
    t = c * NS + s
    zero16 = jnp.zeros((16,), jnp.float32)
    ones16 = jnp.ones((16,), jnp.float32)

    base = pl.multiple_of(t * EPT, 8)
    idma = pltpu.async_copy(edge_hbm.at[pl.ds(E + base, EPT)], didx, isem)

    def z_body(i, carry):
        hloc[pl.ds(i * 16, 16)] = zero16
        return carry

    lax.fori_loop(0, N // 16, z_body, 0)
    idma.wait()

    def e_body(i, carry):
        idx = didx[pl.ds(i * 16, 16)]
        plsc.addupdate_scatter(hloc, [idx], ones16)
        return carry

    lax.fori_loop(0, EPT // 16, e_body, 0)
    pltpu.sync_copy(hloc, shared.at[pl.ds(pl.multiple_of(s * N, 8), N)])
    plsc.subcore_barrier()

    def reduce_stripe(off, size):
        for r in range(NS):
            pltpu.sync_copy(
                shared.at[pl.ds(r * N + off, size)],
                tbuf.at[pl.ds(r * STRIPE, size)],
            )

        def r_body(j, carry):
            acc = tbuf[pl.ds(j * 16, 16)]
            for r in range(1, NS):
                acc = acc + tbuf[pl.ds(r * STRIPE + j * 16, 16)]
            obuf[pl.ds(j * 16, 16)] = acc
            return carry

        lax.fori_loop(0, size // 16, r_body, 0)
        pltpu.sync_copy(
            obuf.at[pl.ds(0, size)], hist_hbm.at[pl.ds(c * N + off, size)]
        )

    _stripe_copy(reduce_stripe)


@functools.partial(
    pl.kernel,
    out_type=jax.ShapeDtypeStruct((2 * N, D), jnp.float32),
    mesh=_MESH,
    scratch_types=[
        pltpu.VMEM((EPT,), jnp.int32),
        pltpu.VMEM((EPT,), jnp.int32),
        pltpu.VMEM((CH, D), jnp.float32),
        pltpu.VMEM((CH, D), jnp.float32),
        pltpu.VMEM_SHARED((N, D), jnp.float32),
        pltpu.SemaphoreType.DMA,
        pltpu.SemaphoreType.DMA,
        pltpu.SemaphoreType.DMA,
    ],
)
def _sc_scatter(edge_hbm, z_hbm, zeros_hbm, out_hbm, sidx, didx,
                rows0, rows1, shared, gsem0, gsem1, isem):
    c = lax.axis_index("c")
    s = lax.axis_index("s")
    t = c * NS + s
    base = pl.multiple_of(t * EPT, 8)
    pltpu.async_copy(edge_hbm.at[pl.ds(base, EPT)], sidx, isem).wait()
    idma = pltpu.async_copy(edge_hbm.at[pl.ds(E + base, EPT)], didx, isem)
    _stripe_copy(
        lambda off, size: pltpu.sync_copy(
            zeros_hbm.at[pl.ds(0, size)], shared.at[pl.ds(off, size)]
        )
    )
    idma.wait()
    plsc.subcore_barrier()

    def gather(i, rows, gsem):
        off = pl.multiple_of(i * CH, 8)
        pltpu.async_copy(z_hbm.at[sidx.at[pl.ds(off, CH)]], rows, gsem)

    def gwait(rows, gsem):
        pltpu.make_async_copy(z_hbm.at[pl.ds(0, CH)], rows, gsem).wait()

    def scatter(i, rows):
        off = pl.multiple_of(i * CH, 8)
        pltpu.sync_copy(rows, shared.at[didx.at[pl.ds(off, CH)]], add=True)

    gather(0, rows0, gsem0)
    gather(1, rows1, gsem1)

    def body(k, carry):
        i0 = 2 * k
        gwait(rows0, gsem0)
        scatter(i0, rows0)
        gather(jnp.minimum(i0 + 2, NITER - 1), rows0, gsem0)
        gwait(rows1, gsem1)
        scatter(i0 + 1, rows1)
        gather(jnp.minimum(i0 + 3, NITER - 1), rows1, gsem1)
        return carry

    lax.fori_loop(0, (NITER - 1) // 2, body, 0)
    gwait(rows0, gsem0)
    scatter(NITER - 1, rows0)
    gwait(rows1, gsem1)
    plsc.subcore_barrier()
    _stripe_copy(
        lambda off, size: pltpu.sync_copy(
            shared.at[pl.ds(off, size)], out_hbm.at[pl.ds(c * N + off, size)]
        )
    )


def _dinv(h0, h1):
    deg = 1.0 + h0[...] + h1[...]
    return lax.rsqrt(deg)


def _tc_a_body(x_ref, w1_ref, h0_ref, h1_ref, z1_ref):
    dinv = _dinv(h0_ref, h1_ref)
    y = jnp.dot(x_ref[...], w1_ref[...], preferred_element_type=jnp.float32)
    z1_ref[...] = y * dinv


def _tc_b_body(p0_ref, p1_ref, z1_ref, h0_ref, h1_ref, w2_ref, b1_ref, z2_ref):
    dinv = _dinv(h0_ref, h1_ref)
    h = dinv * (p0_ref[...] + p1_ref[...] + z1_ref[...]) + b1_ref[...]
    h = jnp.maximum(h, 0.0)
    y = jnp.dot(h, w2_ref[...], preferred_element_type=jnp.float32)
    z2_ref[...] = y * dinv


def _tc_c_body(q0_ref, q1_ref, z2_ref, h0_ref, h1_ref, b2_ref, out_ref):
    dinv = _dinv(h0_ref, h1_ref)
    o = dinv * (q0_ref[...] + q1_ref[...] + z2_ref[...]) + b2_ref[...]
    m = jnp.max(o, axis=1, keepdims=True)
    ex = jnp.exp(o - m)
    lse = jnp.log(jnp.sum(ex, axis=1, keepdims=True)) + m
    out_ref[...] = o - lse


def _row_spec(shift=0, width=D):
    return pl.BlockSpec((BM, width), lambda i, s=shift: (i + s, 0))


def _full_spec(shape):
    return pl.BlockSpec(shape, lambda i: (0, 0))


def kernel(x, edge_index, W1, b1, W2, b2):
    ei = edge_index.reshape(2 * E)
    zeros128 = jnp.zeros((STRIPE, D), jnp.float32)
    b1r = b1.reshape(1, D)
    b2r = b2.reshape(1, D)

    hist = _sc_hist(ei).reshape(2 * N, 1)

    z1 = pl.pallas_call(
        _tc_a_body,
        grid=(GRID,),
        in_specs=[
            _row_spec(),
            _full_spec((D, D)),
            _row_spec(width=1),
            _row_spec(shift=GRID, width=1),
        ],
        out_specs=_row_spec(),
        out_shape=jax.ShapeDtypeStruct((N, D), jnp.float32),
    )(x, W1, hist, hist)

    p = _sc_scatter(ei, z1, zeros128)

    z2 = pl.pallas_call(
        _tc_b_body,
        grid=(GRID,),
        in_specs=[
            _row_spec(),
            _row_spec(shift=GRID),
            _row_spec(),
            _row_spec(width=1),
            _row_spec(shift=GRID, width=1),
            _full_spec((D, D)),
            _full_spec((1, D)),
        ],
        out_specs=_row_spec(),
        out_shape=jax.ShapeDtypeStruct((N, D), jnp.float32),
    )(p, p, z1, hist, hist, W2, b1r)

    q = _sc_scatter(ei, z2, zeros128)

    out = pl.pallas_call(
        _tc_c_body,
        grid=(GRID,),
        in_specs=[
            _row_spec(),
            _row_spec(shift=GRID),
            _row_spec(),
            _row_spec(width=1),
            _row_spec(shift=GRID, width=1),
            _full_spec((1, D)),
        ],
        out_specs=_row_spec(),
        out_shape=jax.ShapeDtypeStruct((N, D), jnp.float32),
    )(q, q, z2, hist, hist, b2r)

    return out


# speedup vs baseline: 1.3392x; 1.0509x over previous
import functools

import jax
import jax.numpy as jnp
from jax import lax
from jax.experimental import pallas as pl
from jax.experimental.pallas import tpu as pltpu
from jax.experimental.pallas import tpu_sc as plsc

N = 10000
E = 320000
D = 128

NC = 2
NS = 16
NW = NC * NS
EPT = E // NW
CH = 80
NITER = EPT // CH

STRIPE = 640
LAST = N - 15 * STRIPE
BM = 2000
GRID = N // BM

_MESH = plsc.VectorSubcoreMesh(
    core_axis_name="c", subcore_axis_name="s", num_cores=NC, num_subcores=NS
)


def _stripe_copy(do_copy):
    s = lax.axis_index("s")

    @pl.when(s < 15)
    def _():
        off = pl.multiple_of(s * STRIPE, 8)
        do_copy(off, STRIPE)

    @pl.when(s == 15)
    def _():
        do_copy(15 * STRIPE, LAST)


@functools.partial(
    pl.kernel,
    out_type=jax.ShapeDtypeStruct((2 * N,), jnp.float32),
    mesh=_MESH,
    scratch_types=[
        pltpu.VMEM((EPT,), jnp.int32),
        pltpu.VMEM((N,), jnp.float32),
        pltpu.VMEM((NS * STRIPE,), jnp.float32),
        pltpu.VMEM((STRIPE,), jnp.float32),
        pltpu.VMEM_SHARED((NS * N,), jnp.float32),
        pltpu.SemaphoreType.DMA,
    ],
    compiler_params=pltpu.CompilerParams(needs_layout_passes=False),
)
def _sc_hist(edge_hbm, hist_hbm, didx, hloc, tbuf, obuf, shared, isem):
    c = lax.axis_index("c")
    s = lax.axis_index("s")
    t = c * NS + s
    zero16 = jnp.zeros((16,), jnp.float32)
    ones16 = jnp.ones((16,), jnp.float32)

    base = pl.multiple_of(t * EPT, 8)
    idma = pltpu.async_copy(edge_hbm.at[pl.ds(E + base, EPT)], didx, isem)

    def z_body(i, carry):
        hloc[pl.ds(i * 16, 16)] = zero16
        return carry

    lax.fori_loop(0, N // 16, z_body, 0)
    idma.wait()

    def e_body(i, carry):
        idx = didx[pl.ds(i * 16, 16)]
        plsc.addupdate_scatter(hloc, [idx], ones16)
        return carry

    lax.fori_loop(0, EPT // 16, e_body, 0)
    pltpu.sync_copy(hloc, shared.at[pl.ds(pl.multiple_of(s * N, 8), N)])
    plsc.subcore_barrier()

    def reduce_stripe(off, size):
        for r in range(NS):
            pltpu.sync_copy(
                shared.at[pl.ds(r * N + off, size)],
                tbuf.at[pl.ds(r * STRIPE, size)],
            )

        def r_body(j, carry):
            acc = tbuf[pl.ds(j * 16, 16)]
            for r in range(1, NS):
                acc = acc + tbuf[pl.ds(r * STRIPE + j * 16, 16)]
            obuf[pl.ds(j * 16, 16)] = acc
            return carry

        lax.fori_loop(0, size // 16, r_body, 0)
        pltpu.sync_copy(
            obuf.at[pl.ds(0, size)], hist_hbm.at[pl.ds(c * N + off, size)]
        )

    _stripe_copy(reduce_stripe)


@functools.partial(
    pl.kernel,
    out_type=jax.ShapeDtypeStruct((2 * N, D), jnp.float32),
    mesh=_MESH,
    scratch_types=[
        pltpu.VMEM((EPT,), jnp.int32),
        pltpu.VMEM((EPT,), jnp.int32),
        pltpu.VMEM((CH, D), jnp.float32),
        pltpu.VMEM((CH, D), jnp.float32),
        pltpu.VMEM_SHARED((N, D), jnp.float32),
        pltpu.SemaphoreType.DMA,
        pltpu.SemaphoreType.DMA,
        pltpu.SemaphoreType.DMA,
    ],
)
def _sc_scatter(edge_hbm, z_hbm, zeros_hbm, out_hbm, sidx, didx,
                rows0, rows1, shared, gsem0, gsem1, isem):
    c = lax.axis_index("c")
    s = lax.axis_index("s")
    t = c * NS + s
    base = pl.multiple_of(t * EPT, 8)
    pltpu.async_copy(edge_hbm.at[pl.ds(base, EPT)], sidx, isem).wait()
    idma = pltpu.async_copy(edge_hbm.at[pl.ds(E + base, EPT)], didx, isem)
    _stripe_copy(
        lambda off, size: pltpu.sync_copy(
            zeros_hbm.at[pl.ds(0, size)], shared.at[pl.ds(off, size)]
        )
    )
    idma.wait()
    plsc.subcore_barrier()

    def gather(i, rows, gsem):
        off = pl.multiple_of(i * CH, 8)
        pltpu.async_copy(z_hbm.at[sidx.at[pl.ds(off, CH)]], rows, gsem)

    def gwait(rows, gsem):
        pltpu.make_async_copy(z_hbm.at[pl.ds(0, CH)], rows, gsem).wait()

    def scatter(i, rows):
        off = pl.multiple_of(i * CH, 8)
        pltpu.sync_copy(rows, shared.at[didx.at[pl.ds(off, CH)]], add=True)

    gather(0, rows0, gsem0)
    gather(1, rows1, gsem1)

    def body(k, carry):
        i0 = 2 * k
        gwait(rows0, gsem0)
        scatter(i0, rows0)
        gather(jnp.minimum(i0 + 2, NITER - 1), rows0, gsem0)
        gwait(rows1, gsem1)
        scatter(i0 + 1, rows1)
        gather(jnp.minimum(i0 + 3, NITER - 1), rows1, gsem1)
        return carry

    lax.fori_loop(0, (NITER - 1) // 2, body, 0)
    gwait(rows0, gsem0)
    scatter(NITER - 1, rows0)
    gwait(rows1, gsem1)
    plsc.subcore_barrier()
    _stripe_copy(
        lambda off, size: pltpu.sync_copy(
            shared.at[pl.ds(off, size)], out_hbm.at[pl.ds(c * N + off, size)]
        )
    )


def _dinv(h0, h1):
    deg = 1.0 + h0[...] + h1[...]
    return lax.rsqrt(deg).reshape(BM, 1)


def _tc_a_body(x_ref, w1_ref, h0_ref, h1_ref, z1_ref):
    dinv = _dinv(h0_ref, h1_ref)
    y = jnp.dot(x_ref[...], w1_ref[...], preferred_element_type=jnp.float32)
    z1_ref[...] = y * dinv


def _tc_b_body(p0_ref, p1_ref, z1_ref, h0_ref, h1_ref, w2_ref, b1_ref, z2_ref):
    dinv = _dinv(h0_ref, h1_ref)
    h = dinv * (p0_ref[...] + p1_ref[...] + z1_ref[...]) + b1_ref[...]
    h = jnp.maximum(h, 0.0)
    y = jnp.dot(h, w2_ref[...], preferred_element_type=jnp.float32)
    z2_ref[...] = y * dinv


def _tc_c_body(q0_ref, q1_ref, z2_ref, h0_ref, h1_ref, b2_ref, out_ref):
    dinv = _dinv(h0_ref, h1_ref)
    o = dinv * (q0_ref[...] + q1_ref[...] + z2_ref[...]) + b2_ref[...]
    m = jnp.max(o, axis=1, keepdims=True)
    ex = jnp.exp(o - m)
    lse = jnp.log(jnp.sum(ex, axis=1, keepdims=True)) + m
    out_ref[...] = o - lse


def _row_spec(shift=0, width=D):
    return pl.BlockSpec((BM, width), lambda i, s=shift: (i + s, 0))


def _h_spec(shift=0):
    return pl.BlockSpec((1, 1, BM), lambda i, s=shift: (i + s, 0, 0))


def _full_spec(shape):
    return pl.BlockSpec(shape, lambda i: (0, 0))


def kernel(x, edge_index, W1, b1, W2, b2):
    ei = edge_index.reshape(2 * E)
    zeros128 = jnp.zeros((STRIPE, D), jnp.float32)
    b1r = b1.reshape(1, D)
    b2r = b2.reshape(1, D)

    hist = _sc_hist(ei).reshape(2 * GRID, 1, BM)

    z1 = pl.pallas_call(
        _tc_a_body,
        grid=(GRID,),
        in_specs=[
            _row_spec(),
            _full_spec((D, D)),
            _h_spec(),
            _h_spec(shift=GRID),
        ],
        out_specs=_row_spec(),
        out_shape=jax.ShapeDtypeStruct((N, D), jnp.float32),
    )(x, W1, hist, hist)

    p = _sc_scatter(ei, z1, zeros128)

    z2 = pl.pallas_call(
        _tc_b_body,
        grid=(GRID,),
        in_specs=[
            _row_spec(),
            _row_spec(shift=GRID),
            _row_spec(),
            _h_spec(),
            _h_spec(shift=GRID),
            _full_spec((D, D)),
            _full_spec((1, D)),
        ],
        out_specs=_row_spec(),
        out_shape=jax.ShapeDtypeStruct((N, D), jnp.float32),
    )(p, p, z1, hist, hist, W2, b1r)

    q = _sc_scatter(ei, z2, zeros128)

    out = pl.pallas_call(
        _tc_c_body,
        grid=(GRID,),
        in_specs=[
            _row_spec(),
            _row_spec(shift=GRID),
            _row_spec(),
            _h_spec(),
            _h_spec(shift=GRID),
            _full_spec((1, D)),
        ],
        out_specs=_row_spec(),
        out_shape=jax.ShapeDtypeStruct((N, D), jnp.float32),
    )(q, q, z2, hist, hist, b2r)

    return out
